# Initial kernel scaffold; baseline (speedup 1.0000x reference)
import jax
import jax.numpy as jnp
from jax.experimental import pallas as pl


def kernel(x, edge_index, batch, W0, a_src0, a_dst0, b0, W1, a_src1, a_dst1, b1, W_lin1, b_lin1, W_lin2, b_lin2):
    raise NotImplementedError("write your pallas kernel here")


# speedup vs baseline: 45.9387x; 45.9387x over previous
import functools

import jax
import jax.numpy as jnp
from jax import lax
from jax.experimental import pallas as pl
from jax.experimental.pallas import tpu as pltpu
from jax.experimental.pallas import tpu_sc as plsc

_N = 10000
_E = 320000
_F = 128
_H = 8
_C = 16
_G = 64
_HID = 16
_NCLS = 10

_CH = 128
_NW = 32
_ET = _E + _N
_K = -(-_ET // (_NW * _CH))
_EPAD = _NW * _CH * _K
_ACC_ROWS = 10112
_RPT = _ACC_ROWS // 16
_ACC_W = 144
_PAD_DST = _N


def _edge_pass(src_idx, dst_idx, h, atab, btab, zacc):
    mesh = plsc.VectorSubcoreMesh(core_axis_name="c", subcore_axis_name="s")

    @functools.partial(
        pl.kernel,
        out_type=jax.ShapeDtypeStruct((2, _ACC_ROWS, _ACC_W), jnp.float32),
        mesh=mesh,
        scratch_types=[
            pltpu.VMEM((_CH,), jnp.int32),
            pltpu.VMEM((_CH,), jnp.int32),
            pltpu.VMEM((_CH, _F), jnp.float32),
            pltpu.VMEM((_CH, 16), jnp.float32),
            pltpu.VMEM((_CH, 16), jnp.float32),
            pltpu.VMEM((_CH, _ACC_W), jnp.float32),
            pltpu.VMEM_SHARED((_ACC_ROWS, _ACC_W), jnp.float32),
            pltpu.SemaphoreType.DMA,
            pltpu.SemaphoreType.DMA,
            pltpu.SemaphoreType.DMA,
        ],
        compiler_params=pltpu.CompilerParams(use_tc_tiling_on_sc=False),
    )
    def k(src_hbm, dst_hbm, h_hbm, a_hbm, b_hbm, z_hbm, out_hbm,
          idx_s, idx_d, hrows, arows, brows, stage, acc, sem1, sem2, sem3):
        c = lax.axis_index("c")
        s = lax.axis_index("s")
        wid = s * 2 + c
        pltpu.sync_copy(z_hbm.at[pl.ds(s * _RPT, _RPT)],
                        acc.at[pl.ds(s * _RPT, _RPT)])
        plsc.subcore_barrier()
        lane = lax.iota(jnp.int32, 16)

        def chunk(kk, carry):
            base = (wid * _K + kk) * _CH
            pltpu.sync_copy(src_hbm.at[pl.ds(base, _CH)], idx_s)
            pltpu.sync_copy(dst_hbm.at[pl.ds(base, _CH)], idx_d)
            cp1 = pltpu.async_copy(h_hbm.at[idx_s], hrows, sem1)
            cp2 = pltpu.async_copy(a_hbm.at[idx_s], arows, sem2)
            cp3 = pltpu.async_copy(b_hbm.at[idx_d], brows, sem3)
            cp1.wait()
            cp2.wait()
            cp3.wait()

            def edge(e, carry2):
                logit = arows[e, :] + brows[e, :]
                lr = jnp.where(logit >= 0.0, logit, 0.2 * logit)
                lr = jnp.where(lane < 8, lr, 0.0)
                w = jnp.exp(lr)
                w = jnp.where(lane < 8, w, 0.0)
                stage[e, pl.ds(_F, 16)] = w
                for j in range(_H):
                    bidx = jnp.full((16,), j, dtype=jnp.int32)
                    wj = w.at[bidx].get(mode="promise_in_bounds")
                    stage[e, pl.ds(j * 16, 16)] = hrows[e, pl.ds(j * 16, 16)] * wj
                return carry2

            lax.fori_loop(0, _CH, edge, 0)
            pltpu.sync_copy(stage, acc.at[idx_d], add=True)
            return carry

        lax.fori_loop(0, _K, chunk, 0)
        plsc.subcore_barrier()
        pltpu.sync_copy(acc.at[pl.ds(s * _RPT, _RPT)],
                        out_hbm.at[c, pl.ds(s * _RPT, _RPT)])

    return k(src_idx, dst_idx, h, atab, btab, zacc)


def _tc_prep(x, W, MA, MB):
    def body(x_ref, w_ref, ma_ref, mb_ref, h_ref, a_ref, b_ref):
        h = jnp.dot(x_ref[...], w_ref[...], preferred_element_type=jnp.float32)
        h_ref[...] = h
        a_ref[...] = jnp.dot(h, ma_ref[...], preferred_element_type=jnp.float32)
        b_ref[...] = jnp.dot(h, mb_ref[...], preferred_element_type=jnp.float32)

    return pl.pallas_call(
        body,
        out_shape=[jax.ShapeDtypeStruct((_N, _F), jnp.float32),
                   jax.ShapeDtypeStruct((_N, 16), jnp.float32),
                   jax.ShapeDtypeStruct((_N, 16), jnp.float32)],
    )(x, W, MA, MB)


def _normalize(acc_ref, b_ref, e16_ref):
    num = acc_ref[0, :_N, :_F] + acc_ref[1, :_N, :_F]
    den = acc_ref[0, :_N, _F:_F + _H] + acc_ref[1, :_N, _F:_F + _H]
    den_e = jnp.dot(den, e16_ref[...], preferred_element_type=jnp.float32)
    xn = num / (den_e + 1e-16) + b_ref[...]
    return jnp.where(xn > 0.0, xn, jnp.exp(jnp.minimum(xn, 0.0)) - 1.0)


def _tc_combine_prep(acc, bvec, E16, W, MA, MB):
    def body(acc_ref, b_ref, e16_ref, w_ref, ma_ref, mb_ref,
             h_ref, a_ref, b2_ref):
        xn = _normalize(acc_ref, b_ref, e16_ref)
        h = jnp.dot(xn, w_ref[...], preferred_element_type=jnp.float32)
        h_ref[...] = h
        a_ref[...] = jnp.dot(h, ma_ref[...], preferred_element_type=jnp.float32)
        b2_ref[...] = jnp.dot(h, mb_ref[...], preferred_element_type=jnp.float32)

    return pl.pallas_call(
        body,
        out_shape=[jax.ShapeDtypeStruct((_N, _F), jnp.float32),
                   jax.ShapeDtypeStruct((_N, 16), jnp.float32),
                   jax.ShapeDtypeStruct((_N, 16), jnp.float32)],
    )(acc, bvec, E16, W, MA, MB)


def _tc_final(acc, bvec, E16, batch2d, Wl1, bl1, Wl2, bl2):
    def body(acc_ref, b_ref, e16_ref, bt_ref, w1_ref, b1_ref, w2_ref, b2_ref,
             lsm_ref, log_ref):
        xn = _normalize(acc_ref, b_ref, e16_ref)
        gi = lax.broadcasted_iota(jnp.int32, (_N, _G), 1)
        oh = (gi == bt_ref[...]).astype(jnp.float32)
        sums = lax.dot_general(oh, xn, (((0,), (0,)), ((), ())),
                               preferred_element_type=jnp.float32)
        ones = jnp.ones((_N, 1), dtype=jnp.float32)
        counts = lax.dot_general(oh, ones, (((0,), (0,)), ((), ())),
                                 preferred_element_type=jnp.float32)
        pooled = sums / jnp.maximum(counts, 1.0)
        z = jnp.dot(pooled, w1_ref[...], preferred_element_type=jnp.float32)
        z = z + b1_ref[...]
        z = jnp.where(z > 0.0, z, jnp.exp(jnp.minimum(z, 0.0)) - 1.0)
        logits = jnp.dot(z, w2_ref[...], preferred_element_type=jnp.float32)
        logits = logits + b2_ref[...]
        m = jnp.max(logits, axis=-1, keepdims=True)
        sh = logits - m
        lsm = sh - jnp.log(jnp.sum(jnp.exp(sh), axis=-1, keepdims=True))
        lsm_ref[...] = lsm
        log_ref[...] = logits

    return pl.pallas_call(
        body,
        out_shape=[jax.ShapeDtypeStruct((_G, _NCLS), jnp.float32),
                   jax.ShapeDtypeStruct((_G, _NCLS), jnp.float32)],
    )(acc, bvec, E16, batch2d, Wl1, bl1, Wl2, bl2)


def _coef_mats(a_src, a_dst):
    rows = jnp.arange(_F)
    head = rows // _C
    MA = jnp.zeros((_F, 2 * _H), dtype=jnp.float32)
    MA = MA.at[rows, head].set(a_src.reshape(-1))
    MA = MA.at[rows, _H + head].set(a_dst.reshape(-1))
    MB = jnp.concatenate([MA[:, _H:], MA[:, :_H]], axis=1)
    return MA, MB


def kernel(x, edge_index, batch, W0, a_src0, a_dst0, b0,
           W1, a_src1, a_dst1, b1, W_lin1, b_lin1, W_lin2, b_lin2):
    loop = jnp.arange(_N, dtype=edge_index.dtype)
    pad = _EPAD - _ET
    src = jnp.concatenate([edge_index[0], loop,
                           jnp.zeros((pad,), dtype=edge_index.dtype)])
    dst = jnp.concatenate([edge_index[1], loop,
                           jnp.full((pad,), _PAD_DST, dtype=edge_index.dtype)])
    MA0, MB0 = _coef_mats(a_src0, a_dst0)
    MA1, MB1 = _coef_mats(a_src1, a_dst1)
    e16 = jnp.zeros((_H, _F), dtype=jnp.float32)
    e16 = e16.at[jnp.arange(_F) // _C, jnp.arange(_F)].set(1.0)
    zacc = jnp.zeros((_ACC_ROWS, _ACC_W), dtype=jnp.float32)
    bpad = jnp.zeros((_ACC_ROWS - _N, 16), dtype=jnp.float32)

    h0, A0, B0 = _tc_prep(x, W0, MA0, MB0)
    acc0 = _edge_pass(src, dst, h0, A0,
                      jnp.concatenate([B0, bpad], axis=0), zacc)
    h1, A1, B1 = _tc_combine_prep(acc0, b0.reshape(1, _F), e16, W1, MA1, MB1)
    acc1 = _edge_pass(src, dst, h1, A1,
                      jnp.concatenate([B1, bpad], axis=0), zacc)
    lsm, logits = _tc_final(acc1, b1.reshape(1, _F), e16,
                            batch.reshape(_N, 1).astype(jnp.int32),
                            W_lin1, b_lin1.reshape(1, _HID),
                            W_lin2, b_lin2.reshape(1, _NCLS))
    return (lsm, logits)


# speedup vs baseline: 53.0240x; 1.1542x over previous
import functools

import jax
import jax.numpy as jnp
from jax import lax
from jax.experimental import pallas as pl
from jax.experimental.pallas import tpu as pltpu
from jax.experimental.pallas import tpu_sc as plsc

_N = 10000
_E = 320000
_F = 128
_H = 8
_C = 16
_G = 64
_HID = 16
_NCLS = 10

_CH = 80
_NW = 32
_ET = _E + _N
_K = 2 * (-(-_ET // (_NW * _CH * 2)))
_EPAD = _NW * _CH * _K
_ACC_ROWS = 10112
_RPT = _ACC_ROWS // 16
_ACC_W = 144
_PAD_DST = _N


def _edge_pass(src_idx, dst_idx, h, atab, btab, zacc):
    mesh = plsc.VectorSubcoreMesh(core_axis_name="c", subcore_axis_name="s")

    @functools.partial(
        pl.kernel,
        out_type=jax.ShapeDtypeStruct((2, _ACC_ROWS, _ACC_W), jnp.float32),
        mesh=mesh,
        scratch_types=[
            pltpu.VMEM((2, _CH), jnp.int32),
            pltpu.VMEM((2, _CH), jnp.int32),
            pltpu.VMEM((2, _CH, _F), jnp.float32),
            pltpu.VMEM((2, _CH, 16), jnp.float32),
            pltpu.VMEM((2, _CH, 16), jnp.float32),
            pltpu.VMEM((_CH, _ACC_W), jnp.float32),
            pltpu.VMEM_SHARED((_ACC_ROWS, _ACC_W), jnp.float32),
            pltpu.SemaphoreType.DMA((2,)),
            pltpu.SemaphoreType.DMA((2,)),
            pltpu.SemaphoreType.DMA((2,)),
            pltpu.SemaphoreType.DMA((2,)),
        ],
        compiler_params=pltpu.CompilerParams(use_tc_tiling_on_sc=False),
    )
    def k(src_hbm, dst_hbm, h_hbm, a_hbm, b_hbm, z_hbm, out_hbm,
          idx_s, idx_d, hrows, arows, brows, stage, acc,
          isem, hsem, asem, bsem):
        c = lax.axis_index("c")
        s = lax.axis_index("s")
        wid = s * 2 + c
        pltpu.sync_copy(z_hbm.at[pl.ds(s * _RPT, _RPT)],
                        acc.at[pl.ds(s * _RPT, _RPT)])
        lane = lax.iota(jnp.int32, 16)

        def idx_start(kk, b):
            base = (wid * _K + kk) * _CH
            pltpu.async_copy(src_hbm.at[pl.ds(base, _CH)], idx_s.at[b],
                             isem.at[b])
            pltpu.async_copy(dst_hbm.at[pl.ds(base, _CH)], idx_d.at[b],
                             isem.at[b])

        def idx_wait(b):
            pltpu.make_async_copy(src_hbm.at[pl.ds(0, _CH)], idx_s.at[b],
                                  isem.at[b]).wait()
            pltpu.make_async_copy(dst_hbm.at[pl.ds(0, _CH)], idx_d.at[b],
                                  isem.at[b]).wait()

        def gather_start(b):
            pltpu.async_copy(h_hbm.at[idx_s.at[b]], hrows.at[b], hsem.at[b])
            pltpu.async_copy(a_hbm.at[idx_s.at[b]], arows.at[b], asem.at[b])
            pltpu.async_copy(b_hbm.at[idx_d.at[b]], brows.at[b], bsem.at[b])

        def gather_wait(b):
            pltpu.make_async_copy(h_hbm.at[idx_s.at[b]], hrows.at[b],
                                  hsem.at[b]).wait()
            pltpu.make_async_copy(a_hbm.at[idx_s.at[b]], arows.at[b],
                                  asem.at[b]).wait()
            pltpu.make_async_copy(b_hbm.at[idx_d.at[b]], brows.at[b],
                                  bsem.at[b]).wait()

        idx_start(0, 0)
        idx_wait(0)
        gather_start(0)
        idx_start(1, 1)
        plsc.subcore_barrier()

        def pair(i, carry):
            for b in range(2):
                kk = 2 * i + b
                nb = 1 - b
                gather_wait(b)
                @pl.when(kk + 1 < _K)
                def _():
                    idx_wait(nb)
                    gather_start(nb)

                def edge(e, carry2):
                    logit = arows[b, e, :] + brows[b, e, :]
                    lr = jnp.where(logit >= 0.0, logit, 0.2 * logit)
                    lr = jnp.where(lane < 8, lr, 0.0)
                    w = jnp.exp(lr)
                    w = jnp.where(lane < 8, w, 0.0)
                    stage[e, pl.ds(_F, 16)] = w
                    for j in range(_H):
                        bidx = jnp.full((16,), j, dtype=jnp.int32)
                        wj = w.at[bidx].get(mode="promise_in_bounds")
                        stage[e, pl.ds(j * 16, 16)] = (
                            hrows[b, e, pl.ds(j * 16, 16)] * wj)
                    return carry2

                lax.fori_loop(0, _CH, edge, 0)
                pltpu.sync_copy(stage, acc.at[idx_d.at[b]], add=True)
                @pl.when(kk + 2 < _K)
                def _():
                    idx_start(kk + 2, b)
            return carry

        lax.fori_loop(0, _K // 2, pair, 0)
        plsc.subcore_barrier()
        pltpu.sync_copy(acc.at[pl.ds(s * _RPT, _RPT)],
                        out_hbm.at[c, pl.ds(s * _RPT, _RPT)])

    return k(src_idx, dst_idx, h, atab, btab, zacc)


def _tc_prep(x, W, MA, MB):
    def body(x_ref, w_ref, ma_ref, mb_ref, h_ref, a_ref, b_ref):
        h = jnp.dot(x_ref[...], w_ref[...], preferred_element_type=jnp.float32)
        h_ref[...] = h
        a_ref[...] = jnp.dot(h, ma_ref[...], preferred_element_type=jnp.float32)
        b_ref[...] = jnp.dot(h, mb_ref[...], preferred_element_type=jnp.float32)

    return pl.pallas_call(
        body,
        out_shape=[jax.ShapeDtypeStruct((_N, _F), jnp.float32),
                   jax.ShapeDtypeStruct((_N, 16), jnp.float32),
                   jax.ShapeDtypeStruct((_N, 16), jnp.float32)],
    )(x, W, MA, MB)


def _normalize(acc_ref, b_ref, e16_ref):
    num = acc_ref[0, :_N, :_F] + acc_ref[1, :_N, :_F]
    den = acc_ref[0, :_N, _F:_F + _H] + acc_ref[1, :_N, _F:_F + _H]
    den_e = jnp.dot(den, e16_ref[...], preferred_element_type=jnp.float32)
    xn = num / (den_e + 1e-16) + b_ref[...]
    return jnp.where(xn > 0.0, xn, jnp.exp(jnp.minimum(xn, 0.0)) - 1.0)


def _tc_combine_prep(acc, bvec, E16, W, MA, MB):
    def body(acc_ref, b_ref, e16_ref, w_ref, ma_ref, mb_ref,
             h_ref, a_ref, b2_ref):
        xn = _normalize(acc_ref, b_ref, e16_ref)
        h = jnp.dot(xn, w_ref[...], preferred_element_type=jnp.float32)
        h_ref[...] = h
        a_ref[...] = jnp.dot(h, ma_ref[...], preferred_element_type=jnp.float32)
        b2_ref[...] = jnp.dot(h, mb_ref[...], preferred_element_type=jnp.float32)

    return pl.pallas_call(
        body,
        out_shape=[jax.ShapeDtypeStruct((_N, _F), jnp.float32),
                   jax.ShapeDtypeStruct((_N, 16), jnp.float32),
                   jax.ShapeDtypeStruct((_N, 16), jnp.float32)],
    )(acc, bvec, E16, W, MA, MB)


def _tc_final(acc, bvec, E16, batch2d, Wl1, bl1, Wl2, bl2):
    def body(acc_ref, b_ref, e16_ref, bt_ref, w1_ref, b1_ref, w2_ref, b2_ref,
             lsm_ref, log_ref):
        xn = _normalize(acc_ref, b_ref, e16_ref)
        gi = lax.broadcasted_iota(jnp.int32, (_N, _G), 1)
        oh = (gi == bt_ref[...]).astype(jnp.float32)
        sums = lax.dot_general(oh, xn, (((0,), (0,)), ((), ())),
                               preferred_element_type=jnp.float32)
        ones = jnp.ones((_N, 1), dtype=jnp.float32)
        counts = lax.dot_general(oh, ones, (((0,), (0,)), ((), ())),
                                 preferred_element_type=jnp.float32)
        pooled = sums / jnp.maximum(counts, 1.0)
        z = jnp.dot(pooled, w1_ref[...], preferred_element_type=jnp.float32)
        z = z + b1_ref[...]
        z = jnp.where(z > 0.0, z, jnp.exp(jnp.minimum(z, 0.0)) - 1.0)
        logits = jnp.dot(z, w2_ref[...], preferred_element_type=jnp.float32)
        logits = logits + b2_ref[...]
        m = jnp.max(logits, axis=-1, keepdims=True)
        sh = logits - m
        lsm = sh - jnp.log(jnp.sum(jnp.exp(sh), axis=-1, keepdims=True))
        lsm_ref[...] = lsm
        log_ref[...] = logits

    return pl.pallas_call(
        body,
        out_shape=[jax.ShapeDtypeStruct((_G, _NCLS), jnp.float32),
                   jax.ShapeDtypeStruct((_G, _NCLS), jnp.float32)],
    )(acc, bvec, E16, batch2d, Wl1, bl1, Wl2, bl2)


def _coef_mats(a_src, a_dst):
    rows = jnp.arange(_F)
    head = rows // _C
    MA = jnp.zeros((_F, 2 * _H), dtype=jnp.float32)
    MA = MA.at[rows, head].set(a_src.reshape(-1))
    MA = MA.at[rows, _H + head].set(a_dst.reshape(-1))
    MB = jnp.concatenate([MA[:, _H:], MA[:, :_H]], axis=1)
    return MA, MB


def kernel(x, edge_index, batch, W0, a_src0, a_dst0, b0,
           W1, a_src1, a_dst1, b1, W_lin1, b_lin1, W_lin2, b_lin2):
    loop = jnp.arange(_N, dtype=edge_index.dtype)
    pad = _EPAD - _ET
    src = jnp.concatenate([edge_index[0], loop,
                           jnp.zeros((pad,), dtype=edge_index.dtype)])
    dst = jnp.concatenate([edge_index[1], loop,
                           jnp.full((pad,), _PAD_DST, dtype=edge_index.dtype)])
    MA0, MB0 = _coef_mats(a_src0, a_dst0)
    MA1, MB1 = _coef_mats(a_src1, a_dst1)
    e16 = jnp.zeros((_H, _F), dtype=jnp.float32)
    e16 = e16.at[jnp.arange(_F) // _C, jnp.arange(_F)].set(1.0)
    zacc = jnp.zeros((_ACC_ROWS, _ACC_W), dtype=jnp.float32)
    bpad = jnp.zeros((_ACC_ROWS - _N, 16), dtype=jnp.float32)

    h0, A0, B0 = _tc_prep(x, W0, MA0, MB0)
    acc0 = _edge_pass(src, dst, h0, A0,
                      jnp.concatenate([B0, bpad], axis=0), zacc)
    h1, A1, B1 = _tc_combine_prep(acc0, b0.reshape(1, _F), e16, W1, MA1, MB1)
    acc1 = _edge_pass(src, dst, h1, A1,
                      jnp.concatenate([B1, bpad], axis=0), zacc)
    lsm, logits = _tc_final(acc1, b1.reshape(1, _F), e16,
                            batch.reshape(_N, 1).astype(jnp.int32),
                            W_lin1, b_lin1.reshape(1, _HID),
                            W_lin2, b_lin2.reshape(1, _NCLS))
    return (lsm, logits)


# speedup vs baseline: 60.9934x; 1.1503x over previous
import functools

import jax
import jax.numpy as jnp
from jax import lax
from jax.experimental import pallas as pl
from jax.experimental.pallas import tpu as pltpu
from jax.experimental.pallas import tpu_sc as plsc

_N = 10000
_E = 320000
_F = 128
_H = 8
_C = 16
_G = 64
_HID = 16
_NCLS = 10

_CH = 64
_NW = 32
_ET = _E + _N
_K = 2 * (-(-_ET // (_NW * _CH * 2)))
_EPAD = _NW * _CH * _K
_ACC_ROWS = 10112
_RPT = _ACC_ROWS // 16
_ACC_W = 144
_PAD_DST = _N


def _edge_pass(src_idx, dst_idx, h, atab, btab, zacc):
    mesh = plsc.VectorSubcoreMesh(core_axis_name="c", subcore_axis_name="s")

    @functools.partial(
        pl.kernel,
        out_type=jax.ShapeDtypeStruct((2, _ACC_ROWS, _ACC_W), jnp.float32),
        mesh=mesh,
        scratch_types=[
            pltpu.VMEM((2, _CH), jnp.int32),
            pltpu.VMEM((4, _CH), jnp.int32),
            pltpu.VMEM((2, _CH, _F), jnp.float32),
            pltpu.VMEM((2, _CH, 16), jnp.float32),
            pltpu.VMEM((2, _CH, 16), jnp.float32),
            pltpu.VMEM((2, _CH, _ACC_W), jnp.float32),
            pltpu.VMEM_SHARED((_ACC_ROWS, _ACC_W), jnp.float32),
            pltpu.SemaphoreType.DMA((2,)),
            pltpu.SemaphoreType.DMA((2,)),
            pltpu.SemaphoreType.DMA((2,)),
            pltpu.SemaphoreType.DMA((2,)),
            pltpu.SemaphoreType.DMA((2,)),
        ],
        compiler_params=pltpu.CompilerParams(use_tc_tiling_on_sc=False),
    )
    def k(src_hbm, dst_hbm, h_hbm, a_hbm, b_hbm, z_hbm, out_hbm,
          idx_s, idx_d, hrows, arows, brows, stage, acc,
          isem, hsem, asem, bsem, ssem):
        c = lax.axis_index("c")
        s = lax.axis_index("s")
        wid = s * 2 + c
        pltpu.sync_copy(z_hbm.at[pl.ds(s * _RPT, _RPT)],
                        acc.at[pl.ds(s * _RPT, _RPT)])
        lane = lax.iota(jnp.int32, 16)

        def idx_start(kk, b, db):
            base = (wid * _K + kk) * _CH
            pltpu.async_copy(src_hbm.at[pl.ds(base, _CH)], idx_s.at[b],
                             isem.at[b])
            pltpu.async_copy(dst_hbm.at[pl.ds(base, _CH)], idx_d.at[db],
                             isem.at[b])

        def idx_wait(b, db):
            pltpu.make_async_copy(src_hbm.at[pl.ds(0, _CH)], idx_s.at[b],
                                  isem.at[b]).wait()
            pltpu.make_async_copy(dst_hbm.at[pl.ds(0, _CH)], idx_d.at[db],
                                  isem.at[b]).wait()

        def gather_start(b, db):
            pltpu.async_copy(h_hbm.at[idx_s.at[b]], hrows.at[b], hsem.at[b])
            pltpu.async_copy(a_hbm.at[idx_s.at[b]], arows.at[b], asem.at[b])
            pltpu.async_copy(b_hbm.at[idx_d.at[db]], brows.at[b], bsem.at[b])

        def gather_wait(b, db):
            pltpu.make_async_copy(h_hbm.at[idx_s.at[b]], hrows.at[b],
                                  hsem.at[b]).wait()
            pltpu.make_async_copy(a_hbm.at[idx_s.at[b]], arows.at[b],
                                  asem.at[b]).wait()
            pltpu.make_async_copy(b_hbm.at[idx_d.at[db]], brows.at[b],
                                  bsem.at[b]).wait()

        def scatter_wait(b, db):
            pltpu.make_async_copy(stage.at[b], acc.at[idx_d.at[db]],
                                  ssem.at[b]).wait()

        idx_start(0, 0, 0)
        idx_wait(0, 0)
        gather_start(0, 0)
        idx_start(1, 1, 1)
        plsc.subcore_barrier()

        def pair(i, carry):
            kk0 = 2 * i
            for b in range(2):
                kk = kk0 + b
                nb = 1 - b
                gather_wait(b, kk % 4)
                @pl.when(kk + 1 < _K)
                def _():
                    idx_wait(nb, (kk + 1) % 4)
                    gather_start(nb, (kk + 1) % 4)

                @pl.when(kk >= 2)
                def _():
                    scatter_wait(b, (kk + 2) % 4)

                def edge(e, carry2):
                    logit = arows[b, e, :] + brows[b, e, :]
                    lr = jnp.where(logit >= 0.0, logit, 0.2 * logit)
                    lr = jnp.where(lane < 8, lr, 0.0)
                    w = jnp.exp(lr)
                    w = jnp.where(lane < 8, w, 0.0)
                    stage[b, e, pl.ds(_F, 16)] = w
                    for j in range(_H):
                        bidx = jnp.full((16,), j, dtype=jnp.int32)
                        wj = w.at[bidx].get(mode="promise_in_bounds")
                        stage[b, e, pl.ds(j * 16, 16)] = (
                            hrows[b, e, pl.ds(j * 16, 16)] * wj)
                    return carry2

                lax.fori_loop(0, _CH, edge, 0, unroll=4)
                pltpu.async_copy(stage.at[b], acc.at[idx_d.at[kk % 4]],
                                 ssem.at[b], add=True)
                @pl.when(kk + 2 < _K)
                def _():
                    idx_start(kk + 2, b, (kk + 2) % 4)
            return carry

        lax.fori_loop(0, _K // 2, pair, 0)
        scatter_wait(0, (_K - 2) % 4)
        scatter_wait(1, (_K - 1) % 4)
        plsc.subcore_barrier()
        pltpu.sync_copy(acc.at[pl.ds(s * _RPT, _RPT)],
                        out_hbm.at[c, pl.ds(s * _RPT, _RPT)])

    return k(src_idx, dst_idx, h, atab, btab, zacc)


def _tc_prep(x, W, MA, MB):
    def body(x_ref, w_ref, ma_ref, mb_ref, h_ref, a_ref, b_ref):
        h = jnp.dot(x_ref[...], w_ref[...], preferred_element_type=jnp.float32)
        h_ref[...] = h
        a_ref[...] = jnp.dot(h, ma_ref[...], preferred_element_type=jnp.float32)
        b_ref[...] = jnp.dot(h, mb_ref[...], preferred_element_type=jnp.float32)

    return pl.pallas_call(
        body,
        out_shape=[jax.ShapeDtypeStruct((_N, _F), jnp.float32),
                   jax.ShapeDtypeStruct((_N, 16), jnp.float32),
                   jax.ShapeDtypeStruct((_N, 16), jnp.float32)],
    )(x, W, MA, MB)


def _normalize(acc_ref, b_ref, e16_ref):
    num = acc_ref[0, :_N, :_F] + acc_ref[1, :_N, :_F]
    den = acc_ref[0, :_N, _F:_F + _H] + acc_ref[1, :_N, _F:_F + _H]
    den_e = jnp.dot(den, e16_ref[...], preferred_element_type=jnp.float32)
    xn = num / (den_e + 1e-16) + b_ref[...]
    return jnp.where(xn > 0.0, xn, jnp.exp(jnp.minimum(xn, 0.0)) - 1.0)


def _tc_combine_prep(acc, bvec, E16, W, MA, MB):
    def body(acc_ref, b_ref, e16_ref, w_ref, ma_ref, mb_ref,
             h_ref, a_ref, b2_ref):
        xn = _normalize(acc_ref, b_ref, e16_ref)
        h = jnp.dot(xn, w_ref[...], preferred_element_type=jnp.float32)
        h_ref[...] = h
        a_ref[...] = jnp.dot(h, ma_ref[...], preferred_element_type=jnp.float32)
        b2_ref[...] = jnp.dot(h, mb_ref[...], preferred_element_type=jnp.float32)

    return pl.pallas_call(
        body,
        out_shape=[jax.ShapeDtypeStruct((_N, _F), jnp.float32),
                   jax.ShapeDtypeStruct((_N, 16), jnp.float32),
                   jax.ShapeDtypeStruct((_N, 16), jnp.float32)],
    )(acc, bvec, E16, W, MA, MB)


def _tc_final(acc, bvec, E16, batch2d, Wl1, bl1, Wl2, bl2):
    def body(acc_ref, b_ref, e16_ref, bt_ref, w1_ref, b1_ref, w2_ref, b2_ref,
             lsm_ref, log_ref):
        xn = _normalize(acc_ref, b_ref, e16_ref)
        gi = lax.broadcasted_iota(jnp.int32, (_N, _G), 1)
        oh = (gi == bt_ref[...]).astype(jnp.float32)
        sums = lax.dot_general(oh, xn, (((0,), (0,)), ((), ())),
                               preferred_element_type=jnp.float32)
        ones = jnp.ones((_N, 1), dtype=jnp.float32)
        counts = lax.dot_general(oh, ones, (((0,), (0,)), ((), ())),
                                 preferred_element_type=jnp.float32)
        pooled = sums / jnp.maximum(counts, 1.0)
        z = jnp.dot(pooled, w1_ref[...], preferred_element_type=jnp.float32)
        z = z + b1_ref[...]
        z = jnp.where(z > 0.0, z, jnp.exp(jnp.minimum(z, 0.0)) - 1.0)
        logits = jnp.dot(z, w2_ref[...], preferred_element_type=jnp.float32)
        logits = logits + b2_ref[...]
        m = jnp.max(logits, axis=-1, keepdims=True)
        sh = logits - m
        lsm = sh - jnp.log(jnp.sum(jnp.exp(sh), axis=-1, keepdims=True))
        lsm_ref[...] = lsm
        log_ref[...] = logits

    return pl.pallas_call(
        body,
        out_shape=[jax.ShapeDtypeStruct((_G, _NCLS), jnp.float32),
                   jax.ShapeDtypeStruct((_G, _NCLS), jnp.float32)],
    )(acc, bvec, E16, batch2d, Wl1, bl1, Wl2, bl2)


def _coef_mats(a_src, a_dst):
    rows = jnp.arange(_F)
    head = rows // _C
    MA = jnp.zeros((_F, 2 * _H), dtype=jnp.float32)
    MA = MA.at[rows, head].set(a_src.reshape(-1))
    MA = MA.at[rows, _H + head].set(a_dst.reshape(-1))
    MB = jnp.concatenate([MA[:, _H:], MA[:, :_H]], axis=1)
    return MA, MB


def kernel(x, edge_index, batch, W0, a_src0, a_dst0, b0,
           W1, a_src1, a_dst1, b1, W_lin1, b_lin1, W_lin2, b_lin2):
    loop = jnp.arange(_N, dtype=edge_index.dtype)
    pad = _EPAD - _ET
    src = jnp.concatenate([edge_index[0], loop,
                           jnp.zeros((pad,), dtype=edge_index.dtype)])
    dst = jnp.concatenate([edge_index[1], loop,
                           jnp.full((pad,), _PAD_DST, dtype=edge_index.dtype)])
    MA0, MB0 = _coef_mats(a_src0, a_dst0)
    MA1, MB1 = _coef_mats(a_src1, a_dst1)
    e16 = jnp.zeros((_H, _F), dtype=jnp.float32)
    e16 = e16.at[jnp.arange(_F) // _C, jnp.arange(_F)].set(1.0)
    zacc = jnp.zeros((_ACC_ROWS, _ACC_W), dtype=jnp.float32)
    bpad = jnp.zeros((_ACC_ROWS - _N, 16), dtype=jnp.float32)

    h0, A0, B0 = _tc_prep(x, W0, MA0, MB0)
    acc0 = _edge_pass(src, dst, h0, A0,
                      jnp.concatenate([B0, bpad], axis=0), zacc)
    h1, A1, B1 = _tc_combine_prep(acc0, b0.reshape(1, _F), e16, W1, MA1, MB1)
    acc1 = _edge_pass(src, dst, h1, A1,
                      jnp.concatenate([B1, bpad], axis=0), zacc)
    lsm, logits = _tc_final(acc1, b1.reshape(1, _F), e16,
                            batch.reshape(_N, 1).astype(jnp.int32),
                            W_lin1, b_lin1.reshape(1, _HID),
                            W_lin2, b_lin2.reshape(1, _NCLS))
    return (lsm, logits)
